# baseline (speedup 1.0000x reference)
import functools

import jax
import jax.numpy as jnp
from jax import lax
from jax.experimental import pallas as pl
from jax.experimental.pallas import tpu as pltpu
from jax.experimental.pallas import tpu_sc as plsc

_DIM = 770
_KQ = 65536
_B = 4096
_EMB = 768
_NW = 32
_C = _KQ // _NW


def _build_body(emb_ref, extra_ref, o_ref):
    o_ref[0:_EMB, :] = emb_ref[...].T
    o_ref[_EMB:_DIM, :] = extra_ref[...]


def _build_emb_t(embedding_batch, extra):
    n = embedding_batch.shape[0]
    return pl.pallas_call(
        _build_body,
        in_specs=[
            pl.BlockSpec((n, _EMB), lambda: (0, 0)),
            pl.BlockSpec((2, _B), lambda: (0, 0)),
        ],
        out_specs=pl.BlockSpec((_DIM, _B), lambda: (0, 0)),
        out_shape=jax.ShapeDtypeStruct((_DIM, _B), jnp.float32),
    )(embedding_batch, extra)


def _sc_body(p_hbm, emb_t_hbm, q_hbm, o_hbm, p_vmem):
    w = lax.axis_index("s") * 2 + lax.axis_index("c")
    pltpu.sync_copy(p_hbm, p_vmem)
    p = pl.multiple_of(p_vmem[...][0], _B)
    c0 = lax.div(p, _C)

    @pl.when(w == 0)
    def _():
        pltpu.sync_copy(
            emb_t_hbm.at[:, pl.ds(0, _C)], o_hbm.at[:, pl.ds(p, _C)]
        )

    @pl.when(w == 1)
    def _():
        pltpu.sync_copy(
            emb_t_hbm.at[:, pl.ds(_C, _C)], o_hbm.at[:, pl.ds(p + _C, _C)]
        )

    @pl.when(w >= 2)
    def _():
        w2 = w - 2
        chunk = jnp.where(w2 < c0, w2, w2 + 2)
        col = pl.multiple_of(chunk * _C, _C)
        pltpu.sync_copy(
            q_hbm.at[:, pl.ds(col, _C)], o_hbm.at[:, pl.ds(col, _C)]
        )


_sc_update = functools.partial(
    pl.kernel,
    out_type=jax.ShapeDtypeStruct((_DIM, _KQ), jnp.float32),
    mesh=plsc.VectorSubcoreMesh(core_axis_name="c", subcore_axis_name="s"),
    scratch_types=[pltpu.VMEM((16,), jnp.int32)],
)(_sc_body)


def kernel(embedding_batch, CLabel, NumofLabel, queue, queue_ptr):
    n = embedding_batch.shape[0]
    idx = jnp.arange(n, dtype=jnp.float32) + (
        jnp.asarray(NumofLabel, dtype=jnp.float32) - jnp.float32(n)
    )
    extra = jnp.stack([CLabel.astype(jnp.float32), idx])

    ptr = queue_ptr[0]
    p = jnp.where(ptr + _B >= _KQ - 1, jnp.int32(0), ptr).astype(jnp.int32)
    p_arr = jnp.full((16,), p, dtype=jnp.int32)

    emb_t = _build_emb_t(embedding_batch, extra)
    out = _sc_update(p_arr, emb_t, queue)

    new_ptr = p + jnp.int32(_B)
    return (out, new_ptr)


# speedup vs baseline: 45.1082x; 45.1082x over previous
import jax
import jax.numpy as jnp
from jax.experimental import pallas as pl
from jax.experimental.pallas import tpu as pltpu

_DIM = 770
_KQ = 65536
_B = 4096
_EMB = 768
_R = 32


def _body(p_ref, emb_ref, extra_ref, q_ref, o_ref, scr_ref):
    i = pl.program_id(0)

    @pl.when(i == 0)
    def _():
        scr_ref[...] = emb_ref[...].T

    o_ref[...] = q_ref[...]
    p = pl.multiple_of(p_ref[0], _B)
    r0 = pl.multiple_of(jnp.minimum(i * _R, _EMB - _R), _R)
    emb_t = scr_ref[pl.ds(r0, _R), :]
    rows = jax.lax.broadcasted_iota(jnp.int32, (_R, 1), 0) + i * _R
    vals = jnp.where(rows < _EMB, emb_t, extra_ref[...])
    o_ref[:, pl.ds(p, _B)] = vals


def kernel(embedding_batch, CLabel, NumofLabel, queue, queue_ptr):
    n = embedding_batch.shape[0]
    idx = jnp.arange(n, dtype=jnp.float32) + (
        jnp.asarray(NumofLabel, dtype=jnp.float32) - jnp.float32(n)
    )
    extra = jnp.zeros((_R, _B), dtype=jnp.float32)
    extra = extra.at[0].set(CLabel.astype(jnp.float32))
    extra = extra.at[1].set(idx)

    ptr = queue_ptr[0]
    p = jnp.where(ptr + _B >= _KQ - 1, jnp.int32(0), ptr).astype(jnp.int32)
    p_arr = p.reshape(1)

    nblocks = pl.cdiv(_DIM, _R)
    emb_blocks = _EMB // _R

    out = pl.pallas_call(
        _body,
        grid=(nblocks,),
        in_specs=[
            pl.BlockSpec(memory_space=pltpu.SMEM),
            pl.BlockSpec((n, _EMB), lambda i: (0, 0)),
            pl.BlockSpec((_R, _B), lambda i: (0, 0)),
            pl.BlockSpec((_R, _KQ), lambda i: (i, 0)),
        ],
        out_specs=pl.BlockSpec((_R, _KQ), lambda i: (i, 0)),
        out_shape=jax.ShapeDtypeStruct((_DIM, _KQ), jnp.float32),
        scratch_shapes=[pltpu.VMEM((_EMB, _B), jnp.float32)],
        compiler_params=pltpu.CompilerParams(
            dimension_semantics=("arbitrary",),
        ),
    )(p_arr, embedding_batch, extra, queue)

    new_ptr = p + jnp.int32(_B)
    return (out, new_ptr)
